# baseline (speedup 1.0000x reference)
import jax
import jax.numpy as jnp
import numpy as np
from jax.experimental import pallas as pl
from jax.experimental.pallas import tpu as pltpu

_GRID = 24
_BW = 2
_TOKENS = 577
_BH = 48
_NT = 4
_ROWS = _NT * _BH


def _border_cols() -> np.ndarray:
    cols = []
    for r in range(_GRID):
        for c in range(_GRID):
            if r < _BW or r >= _GRID - _BW or c < _BW or c >= _GRID - _BW:
                cols.append(1 + r * _GRID + c)
    return np.asarray(sorted(cols), dtype=np.int32)


_COLS = _border_cols()
_NIDX = _COLS.size
_SCALE = np.float32(0.1 / (_NT * _BH * _NIDX))
_MASK = np.zeros((1, _TOKENS), dtype=np.float32)
_MASK[0, _COLS] = 1.0


def _border_mean(p_ref, q_ref, m_ref, o_ref):
    m = m_ref[...]
    o_ref[0, 0] = (jnp.sum(p_ref[...] * m)
                   + jnp.sum(q_ref[...] * m)) * _SCALE


def kernel(attn_0, attn_1, attn_2, attn_3):
    p01 = jnp.stack([attn_0[:, :, 0, :], attn_1[:, :, 0, :]])
    p23 = jnp.stack([attn_2[:, :, 0, :], attn_3[:, :, 0, :]])
    total = pl.pallas_call(
        _border_mean,
        out_shape=jax.ShapeDtypeStruct((1, 1), jnp.float32),
        out_specs=pl.BlockSpec(memory_space=pltpu.SMEM),
    )(p01, p23, jnp.asarray(_MASK))
    return total[0, 0]


# speedup vs baseline: 1.4739x; 1.4739x over previous
import jax
import jax.numpy as jnp
import numpy as np
from jax.experimental import pallas as pl
from jax.experimental.pallas import tpu as pltpu

_GRID = 24
_BW = 2
_TOKENS = 577
_BH = 48
_NT = 4
_ROWS = _NT * _BH


def _border_cols() -> np.ndarray:
    cols = []
    for r in range(_GRID):
        for c in range(_GRID):
            if r < _BW or r >= _GRID - _BW or c < _BW or c >= _GRID - _BW:
                cols.append(1 + r * _GRID + c)
    return np.asarray(sorted(cols), dtype=np.int32)


_COLS = _border_cols()
_NIDX = _COLS.size
_SCALE = np.float32(0.1 / (_NT * _BH * _NIDX))
_MASK = np.zeros((1, _TOKENS), dtype=np.float32)
_MASK[0, _COLS] = 1.0


def _border_mean(r_ref, m_ref, o_ref):
    x = r_ref[...]
    o_ref[0, 0] = jnp.sum(x * m_ref[...]) * _SCALE


def kernel(attn_0, attn_1, attn_2, attn_3):
    rows = jnp.reshape(
        jnp.stack([a[:, :, 0, :] for a in (attn_0, attn_1, attn_2, attn_3)]),
        (_ROWS, _TOKENS))
    total = pl.pallas_call(
        _border_mean,
        out_shape=jax.ShapeDtypeStruct((1, 1), jnp.float32),
        out_specs=pl.BlockSpec(memory_space=pltpu.SMEM),
    )(rows, jnp.asarray(_MASK))
    return total[0, 0]
